# Initial kernel scaffold; baseline (speedup 1.0000x reference)
import jax
import jax.numpy as jnp
from jax.experimental import pallas as pl


def kernel(protein_seq, compound_x, seq_W, seq_b, c1_Wl, c1_Wr, c1_b, c2_Wl, c2_Wr, c2_b, c3_Wl, c3_Wr, c3_b, p1_Wl, p1_Wr, p1_b, p2_Wl, p2_Wr, p2_b, p3_Wl, p3_Wr, p3_b, cfc_W, cfc_b, pfc_W, pfc_b, fc1_W, fc1_b, out_W, out_b, compound_edge_index, compound_batch, protein_edge_index, protein_batch):
    raise NotImplementedError("write your pallas kernel here")


# speedup vs baseline: 8.7890x; 8.7890x over previous
import functools

import jax
import jax.numpy as jnp
from jax import lax
from jax.experimental import pallas as pl
from jax.experimental.pallas import tpu as pltpu
from jax.experimental.pallas import tpu_sc as plsc

NC_NODES = 10000
NP_NODES = 16384
NTOT = NC_NODES + NP_NODES
NACC = 26624
EC = 320000
EP = 524288
E = EC + EP
NW = 32
CHUNK = 128
NCH = 208
EPAD = NW * NCH * CHUNK
B = 8
L = 2048


def _leaky(x):
    return jnp.where(x >= 0, x, 0.01 * x)


def _dot(a, b):
    return jnp.dot(a, b, preferred_element_type=jnp.float32)


def _make_segsum():
    D = 32
    mesh = plsc.VectorSubcoreMesh(core_axis_name="c", subcore_axis_name="s")
    rows_per_tile = NACC // 16
    cp_ch = rows_per_tile // CHUNK

    @functools.partial(
        pl.kernel,
        mesh=mesh,
        compiler_params=pltpu.CompilerParams(use_tc_tiling_on_sc=False),
        out_type=jax.ShapeDtypeStruct((2, NACC, D), jnp.float32),
        scratch_types=[
            pltpu.VMEM((NCH, CHUNK), jnp.int32),
            pltpu.VMEM((NCH, CHUNK), jnp.int32),
            pltpu.VMEM((CHUNK, D), jnp.float32),
            pltpu.VMEM_SHARED((NACC, D), jnp.float32),
            pltpu.SemaphoreType.DMA,
        ],
    )
    def seg(table_h, src_h, dst_h, out_h, sidx, didx, rows, acc, gsem):
        cid = lax.axis_index("c")
        sid = lax.axis_index("s")
        wid = sid * 2 + cid
        base = sid * rows_per_tile

        z16 = jnp.zeros((16,), jnp.float32)

        def zrow(r, _):
            for c in range(D // 16):
                rows[r, pl.ds(c * 16, 16)] = z16
            return 0

        lax.fori_loop(0, CHUNK, zrow, 0)

        def zcp(i, _):
            pltpu.sync_copy(rows, acc.at[pl.ds(base + i * CHUNK, CHUNK)])
            return 0

        lax.fori_loop(0, cp_ch, zcp, 0)

        pltpu.sync_copy(src_h.at[wid], sidx)
        pltpu.sync_copy(dst_h.at[wid], didx)
        plsc.subcore_barrier()

        def body(j, _):
            pltpu.async_copy(table_h.at[sidx.at[j]], rows, gsem).wait()
            pltpu.sync_copy(rows, acc.at[didx.at[j]], add=True)
            return 0

        lax.fori_loop(0, NCH, body, 0)
        plsc.subcore_barrier()

        def ocp(i, _):
            off = base + i * CHUNK
            pltpu.sync_copy(acc.at[pl.ds(off, CHUNK)], rows)
            pltpu.sync_copy(rows, out_h.at[cid, pl.ds(off, CHUNK)])
            return 0

        lax.fori_loop(0, cp_ch, ocp, 0)

    return seg


_segsum32 = _make_segsum()


def _psum(s):
    def body(s_r, o_r):
        o_r[...] = s_r[0] + s_r[1]

    sp = s.reshape(2, (NACC * 32) // 128, 128)
    return pl.pallas_call(
        body,
        out_shape=jax.ShapeDtypeStruct(((NACC * 32) // 128, 128), jnp.float32),
    )(sp).reshape(NACC, 32)


def _dense_a(xpad, cx, wcat3, seqb, cwl1, cwr1, cb1, pwl1, pwr1, pb1):
    def body(xpad_r, cx_r, wcat_r, seqb_r, cwl_r, cwr_r, cb_r, pwl_r, pwr_r,
             pb_r, t1_r, r1_r):
        cxv = cx_r[...]
        t1_r[0:NC_NODES, :] = _dot(cxv, cwl_r[...])
        r1_r[0:NC_NODES, :] = _dot(cxv, cwr_r[...]) + cb_r[...]
        for b in range(B):
            acc = seqb_r[...] * jnp.ones((L, 1), jnp.float32)
            for k in range(5):
                acc = acc + _dot(xpad_r[b, k:k + L, :], wcat_r[k])
            lo = NC_NODES + b * L
            t1_r[lo:lo + L, :] = _dot(acc, pwl_r[...])
            r1_r[lo:lo + L, :] = _dot(acc, pwr_r[...]) + pb_r[...]

    return pl.pallas_call(
        body,
        out_shape=(jax.ShapeDtypeStruct((NTOT, 32), jnp.float32),
                   jax.ShapeDtypeStruct((NTOT, 32), jnp.float32)),
    )(xpad, cx, wcat3, seqb, cwl1, cwr1, cb1, pwl1, pwr1, pb1)


def _combine1(u, r1):
    def body(u_r, r1_r, t_r):
        t_r[...] = _leaky(u_r[0:NTOT, :] + r1_r[...])

    return pl.pallas_call(
        body,
        out_shape=jax.ShapeDtypeStruct((NTOT, 32), jnp.float32),
    )(u, r1)


def _rproj(x, cw, cb, pw, pb, d_out):
    def body(x_r, cw_r, cb_r, pw_r, pb_r, o_r):
        o_r[0:NC_NODES, :] = _dot(x_r[0:NC_NODES], cw_r[...]) + cb_r[...]
        o_r[NC_NODES:NTOT, :] = _dot(x_r[NC_NODES:NTOT], pw_r[...]) + pb_r[...]

    return pl.pallas_call(
        body,
        out_shape=jax.ShapeDtypeStruct((NTOT, d_out), jnp.float32),
    )(x, cw, cb, pw, pb)


def _combine2(u, r2, cwl, pwl):
    def body(u_r, r2_r, cwl_r, pwl_r, o_r):
        o_r[0:NC_NODES, :] = _leaky(
            _dot(u_r[0:NC_NODES], cwl_r[...]) + r2_r[0:NC_NODES])
        o_r[NC_NODES:NTOT, :] = _leaky(
            _dot(u_r[NC_NODES:NTOT], pwl_r[...]) + r2_r[NC_NODES:NTOT])

    return pl.pallas_call(
        body,
        out_shape=jax.ShapeDtypeStruct((NTOT, 64), jnp.float32),
    )(u, r2, cwl, pwl)


def _split3(x2, cw, cb, pw, pb):
    def body(x_r, cw_r, cb_r, pw_r, pb_r, ta_r, tb_r, r3_r):
        xv = x_r[...]
        ta_r[...] = xv[:, 0:32]
        tb_r[...] = xv[:, 32:64]
        r3_r[0:NC_NODES, :] = _dot(xv[0:NC_NODES], cw_r[...]) + cb_r[...]
        r3_r[NC_NODES:NTOT, :] = _dot(xv[NC_NODES:NTOT], pw_r[...]) + pb_r[...]

    return pl.pallas_call(
        body,
        out_shape=(jax.ShapeDtypeStruct((NTOT, 32), jnp.float32),
                   jax.ShapeDtypeStruct((NTOT, 32), jnp.float32),
                   jax.ShapeDtypeStruct((NTOT, 128), jnp.float32)),
    )(x2, cw, cb, pw, pb)


def _combine3(ua, ub, r3, wla, wlb, n):
    def body(ua_r, ub_r, r3_r, wla_r, wlb_r, o_r):
        o_r[...] = _leaky(_dot(ua_r[...], wla_r[...])
                          + _dot(ub_r[...], wlb_r[...]) + r3_r[...])

    return pl.pallas_call(
        body,
        out_shape=jax.ShapeDtypeStruct((n, 128), jnp.float32),
    )(ua, ub, r3, wla, wlb)


def _pool(x, batch, n, nblk):
    bs = n // nblk

    def body(x_r, b_r, o_r):
        ninf = jnp.float32(-jnp.inf)
        pools = [jnp.full((1, 128), ninf, jnp.float32) for _ in range(B)]
        for blk in range(nblk):
            xb = x_r[blk * bs:(blk + 1) * bs, :]
            bb = b_r[blk * bs:(blk + 1) * bs, :]
            for g in range(B):
                m = jnp.max(jnp.where(bb == g, xb, ninf), axis=0,
                            keepdims=True)
                pools[g] = jnp.maximum(pools[g], m)
        o_r[...] = jnp.concatenate(pools, axis=0)

    return pl.pallas_call(
        body,
        out_shape=jax.ShapeDtypeStruct((B, 128), jnp.float32),
    )(x, batch)


def _head(cp, pp, cfc, cfcb, pfc, pfcb, fc1, fc1b, outw, outb):
    def body(cp_r, pp_r, cfc_r, cfcb_r, pfc_r, pfcb_r, fc1_r, fc1b_r, ow_r,
             ob_r, o_r):
        c = _leaky(_dot(cp_r[...], cfc_r[...]) + cfcb_r[...])
        p = _leaky(_dot(pp_r[...], pfc_r[...]) + pfcb_r[...])
        x = jnp.concatenate([c, p], axis=1)
        x = _leaky(_dot(x, fc1_r[...]) + fc1b_r[...])
        o_r[...] = _dot(x, ow_r[...]) + ob_r[...]

    return pl.pallas_call(
        body,
        out_shape=jax.ShapeDtypeStruct((B, 1), jnp.float32),
    )(cp, pp, cfc, cfcb, pfc, pfcb, fc1, fc1b, outw, outb)


def kernel(protein_seq, compound_x, seq_W, seq_b, c1_Wl, c1_Wr, c1_b, c2_Wl,
           c2_Wr, c2_b, c3_Wl, c3_Wr, c3_b, p1_Wl, p1_Wr, p1_b, p2_Wl, p2_Wr,
           p2_b, p3_Wl, p3_Wr, p3_b, cfc_W, cfc_b, pfc_W, pfc_b, fc1_W,
           fc1_b, out_W, out_b, compound_edge_index, compound_batch,
           protein_edge_index, protein_batch):
    xpad = jnp.pad(protein_seq, ((0, 0), (2, 2), (0, 0)))
    wcat3 = seq_W.transpose(2, 1, 0)
    pad = EPAD - E
    j = jnp.arange(pad, dtype=jnp.int32)
    pad_src = (j * 97) % NTOT
    pad_dst = NTOT + (j % (NACC - NTOT))
    src = jnp.concatenate(
        [compound_edge_index[0], protein_edge_index[0] + NC_NODES, pad_src]
    ).reshape(NW, NCH, CHUNK)
    dst = jnp.concatenate(
        [compound_edge_index[1], protein_edge_index[1] + NC_NODES, pad_dst]
    ).reshape(NW, NCH, CHUNK)

    rs = lambda v: v.reshape(1, -1)

    t1, r1 = _dense_a(xpad, compound_x, wcat3, rs(seq_b), c1_Wl.T, c1_Wr.T,
                      rs(c1_b), p1_Wl.T, p1_Wr.T, rs(p1_b))
    u1 = _psum(_segsum32(t1, src, dst))

    t2 = _combine1(u1, r1)
    r2 = _rproj(t2, c2_Wr.T, rs(c2_b), p2_Wr.T, rs(p2_b), 64)
    u2 = _psum(_segsum32(t2, src, dst))

    x2 = _combine2(u2, r2, c2_Wl.T, p2_Wl.T)
    t3a, t3b, r3 = _split3(x2, c3_Wr.T, rs(c3_b), p3_Wr.T, rs(p3_b))
    ua = _psum(_segsum32(t3a, src, dst))
    ub = _psum(_segsum32(t3b, src, dst))

    cwl3 = c3_Wl.T
    pwl3 = p3_Wl.T
    xc3 = _combine3(ua[0:NC_NODES], ub[0:NC_NODES], r3[0:NC_NODES],
                    cwl3[0:32], cwl3[32:64], NC_NODES)
    xp3 = _combine3(ua[NC_NODES:NTOT], ub[NC_NODES:NTOT], r3[NC_NODES:NTOT],
                    pwl3[0:32], pwl3[32:64], NP_NODES)
    cp = _pool(xc3, compound_batch.reshape(-1, 1), NC_NODES, 8)
    pp = _pool(xp3, protein_batch.reshape(-1, 1), NP_NODES, 8)
    return _head(cp, pp, cfc_W.T, rs(cfc_b), pfc_W.T, rs(pfc_b),
                 fc1_W.T, rs(fc1_b), out_W.T, rs(out_b))


# speedup vs baseline: 13.3471x; 1.5186x over previous
import functools

import jax
import jax.numpy as jnp
from jax import lax
from jax.experimental import pallas as pl
from jax.experimental.pallas import tpu as pltpu
from jax.experimental.pallas import tpu_sc as plsc

NC_NODES = 10000
NP_NODES = 16384
NTOT = NC_NODES + NP_NODES
NACC = 26624
EC = 320000
EP = 524288
E = EC + EP
NW = 32
CHUNK = 128
NCH = 208
EPAD = NW * NCH * CHUNK
B = 8
L = 2048


def _leaky(x):
    return jnp.where(x >= 0, x, 0.01 * x)


def _dot(a, b):
    return jnp.dot(a, b, preferred_element_type=jnp.float32)


def _make_segsum():
    D = 32
    mesh = plsc.VectorSubcoreMesh(core_axis_name="c", subcore_axis_name="s")
    rows_per_tile = NACC // 16
    cp_ch = rows_per_tile // CHUNK

    NB = 2
    NGRP = NCH // (2 * NB)

    @functools.partial(
        pl.kernel,
        mesh=mesh,
        compiler_params=pltpu.CompilerParams(use_tc_tiling_on_sc=False),
        out_type=jax.ShapeDtypeStruct((2, NACC, D), jnp.float32),
        scratch_types=[
            pltpu.VMEM((NCH, CHUNK), jnp.int32),
            pltpu.VMEM((NCH, CHUNK), jnp.int32),
            [pltpu.VMEM((CHUNK, D), jnp.float32) for _ in range(2 * NB)],
            pltpu.VMEM_SHARED((NACC, D), jnp.float32),
            [pltpu.SemaphoreType.DMA for _ in range(4)],
        ],
    )
    def seg(table_h, src_h, dst_h, out_h, sidx, didx, rows, acc, sems):
        cid = lax.axis_index("c")
        sid = lax.axis_index("s")
        wid = sid * 2 + cid
        base = sid * rows_per_tile
        bank = [rows[:NB], rows[NB:]]
        gsem = sems[:2]
        ssem = sems[2:]

        z16 = jnp.zeros((16,), jnp.float32)

        def zrow(r, _):
            for c in range(D // 16):
                rows[0][r, pl.ds(c * 16, 16)] = z16
            return 0

        lax.fori_loop(0, CHUNK, zrow, 0)

        def zcp(i, _):
            pltpu.sync_copy(rows[0], acc.at[pl.ds(base + i * CHUNK, CHUNK)])
            return 0

        lax.fori_loop(0, cp_ch, zcp, 0)

        pltpu.sync_copy(src_h.at[wid], sidx)
        pltpu.sync_copy(dst_h.at[wid], didx)
        plsc.subcore_barrier()

        def g_start(j, k, b):
            pltpu.async_copy(table_h.at[sidx.at[j]], bank[k][b], gsem[k])

        def g_wait(k, b):
            pltpu.make_async_copy(table_h.at[sidx.at[0]], bank[k][b],
                                  gsem[k]).wait()

        def s_start(j, k, b):
            pltpu.async_copy(bank[k][b], acc.at[didx.at[j]], ssem[k],
                             add=True)

        def s_wait(k, b):
            pltpu.make_async_copy(bank[k][b], acc.at[didx.at[0]],
                                  ssem[k]).wait()

        for b in range(NB):
            g_start(b, 0, b)

        def body(i, _):
            j0 = i * 2 * NB
            for b in range(NB):
                g_wait(0, b)
            for b in range(NB):
                g_start(j0 + NB + b, 1, b)
            for b in range(NB):
                s_start(j0 + b, 0, b)
            for b in range(NB):
                s_wait(0, b)
            for b in range(NB):
                g_wait(1, b)

            @pl.when(i < NGRP - 1)
            def _():
                for b in range(NB):
                    g_start(j0 + 2 * NB + b, 0, b)

            for b in range(NB):
                s_start(j0 + NB + b, 1, b)
            for b in range(NB):
                s_wait(1, b)
            return 0

        lax.fori_loop(0, NGRP, body, 0)
        plsc.subcore_barrier()

        def ocp(i, _):
            off = base + i * CHUNK
            pltpu.sync_copy(acc.at[pl.ds(off, CHUNK)], rows[0])
            pltpu.sync_copy(rows[0], out_h.at[cid, pl.ds(off, CHUNK)])
            return 0

        lax.fori_loop(0, cp_ch, ocp, 0)

    return seg


_segsum32 = _make_segsum()


def _psum(s):
    def body(s_r, o_r):
        o_r[...] = s_r[0] + s_r[1]

    sp = s.reshape(2, (NACC * 32) // 128, 128)
    return pl.pallas_call(
        body,
        out_shape=jax.ShapeDtypeStruct(((NACC * 32) // 128, 128), jnp.float32),
    )(sp).reshape(NACC, 32)


def _dense_a(xpad, cx, wcat3, seqb, cwl1, cwr1, cb1, pwl1, pwr1, pb1):
    def body(xpad_r, cx_r, wcat_r, seqb_r, cwl_r, cwr_r, cb_r, pwl_r, pwr_r,
             pb_r, t1_r, r1_r):
        cxv = cx_r[...]
        t1_r[0:NC_NODES, :] = _dot(cxv, cwl_r[...])
        r1_r[0:NC_NODES, :] = _dot(cxv, cwr_r[...]) + cb_r[...]
        for b in range(B):
            acc = seqb_r[...] * jnp.ones((L, 1), jnp.float32)
            for k in range(5):
                acc = acc + _dot(xpad_r[b, k:k + L, :], wcat_r[k])
            lo = NC_NODES + b * L
            t1_r[lo:lo + L, :] = _dot(acc, pwl_r[...])
            r1_r[lo:lo + L, :] = _dot(acc, pwr_r[...]) + pb_r[...]

    return pl.pallas_call(
        body,
        out_shape=(jax.ShapeDtypeStruct((NTOT, 32), jnp.float32),
                   jax.ShapeDtypeStruct((NTOT, 32), jnp.float32)),
    )(xpad, cx, wcat3, seqb, cwl1, cwr1, cb1, pwl1, pwr1, pb1)


def _combine1(u, r1):
    def body(u_r, r1_r, t_r):
        t_r[...] = _leaky(u_r[0:NTOT, :] + r1_r[...])

    return pl.pallas_call(
        body,
        out_shape=jax.ShapeDtypeStruct((NTOT, 32), jnp.float32),
    )(u, r1)


def _rproj(x, cw, cb, pw, pb, d_out):
    def body(x_r, cw_r, cb_r, pw_r, pb_r, o_r):
        o_r[0:NC_NODES, :] = _dot(x_r[0:NC_NODES], cw_r[...]) + cb_r[...]
        o_r[NC_NODES:NTOT, :] = _dot(x_r[NC_NODES:NTOT], pw_r[...]) + pb_r[...]

    return pl.pallas_call(
        body,
        out_shape=jax.ShapeDtypeStruct((NTOT, d_out), jnp.float32),
    )(x, cw, cb, pw, pb)


def _combine2(u, r2, cwl, pwl):
    def body(u_r, r2_r, cwl_r, pwl_r, o_r):
        o_r[0:NC_NODES, :] = _leaky(
            _dot(u_r[0:NC_NODES], cwl_r[...]) + r2_r[0:NC_NODES])
        o_r[NC_NODES:NTOT, :] = _leaky(
            _dot(u_r[NC_NODES:NTOT], pwl_r[...]) + r2_r[NC_NODES:NTOT])

    return pl.pallas_call(
        body,
        out_shape=jax.ShapeDtypeStruct((NTOT, 64), jnp.float32),
    )(u, r2, cwl, pwl)


def _split3(x2, cw, cb, pw, pb):
    def body(x_r, cw_r, cb_r, pw_r, pb_r, ta_r, tb_r, r3_r):
        xv = x_r[...]
        ta_r[...] = xv[:, 0:32]
        tb_r[...] = xv[:, 32:64]
        r3_r[0:NC_NODES, :] = _dot(xv[0:NC_NODES], cw_r[...]) + cb_r[...]
        r3_r[NC_NODES:NTOT, :] = _dot(xv[NC_NODES:NTOT], pw_r[...]) + pb_r[...]

    return pl.pallas_call(
        body,
        out_shape=(jax.ShapeDtypeStruct((NTOT, 32), jnp.float32),
                   jax.ShapeDtypeStruct((NTOT, 32), jnp.float32),
                   jax.ShapeDtypeStruct((NTOT, 128), jnp.float32)),
    )(x2, cw, cb, pw, pb)


def _combine3(ua, ub, r3, wla, wlb, n):
    def body(ua_r, ub_r, r3_r, wla_r, wlb_r, o_r):
        o_r[...] = _leaky(_dot(ua_r[...], wla_r[...])
                          + _dot(ub_r[...], wlb_r[...]) + r3_r[...])

    return pl.pallas_call(
        body,
        out_shape=jax.ShapeDtypeStruct((n, 128), jnp.float32),
    )(ua, ub, r3, wla, wlb)


def _pool(x, batch, n, nblk):
    bs = n // nblk

    def body(x_r, b_r, o_r):
        ninf = jnp.float32(-jnp.inf)
        pools = [jnp.full((1, 128), ninf, jnp.float32) for _ in range(B)]
        for blk in range(nblk):
            xb = x_r[blk * bs:(blk + 1) * bs, :]
            bb = b_r[blk * bs:(blk + 1) * bs, :]
            for g in range(B):
                m = jnp.max(jnp.where(bb == g, xb, ninf), axis=0,
                            keepdims=True)
                pools[g] = jnp.maximum(pools[g], m)
        o_r[...] = jnp.concatenate(pools, axis=0)

    return pl.pallas_call(
        body,
        out_shape=jax.ShapeDtypeStruct((B, 128), jnp.float32),
    )(x, batch)


def _head(cp, pp, cfc, cfcb, pfc, pfcb, fc1, fc1b, outw, outb):
    def body(cp_r, pp_r, cfc_r, cfcb_r, pfc_r, pfcb_r, fc1_r, fc1b_r, ow_r,
             ob_r, o_r):
        c = _leaky(_dot(cp_r[...], cfc_r[...]) + cfcb_r[...])
        p = _leaky(_dot(pp_r[...], pfc_r[...]) + pfcb_r[...])
        x = jnp.concatenate([c, p], axis=1)
        x = _leaky(_dot(x, fc1_r[...]) + fc1b_r[...])
        o_r[...] = _dot(x, ow_r[...]) + ob_r[...]

    return pl.pallas_call(
        body,
        out_shape=jax.ShapeDtypeStruct((B, 1), jnp.float32),
    )(cp, pp, cfc, cfcb, pfc, pfcb, fc1, fc1b, outw, outb)


def kernel(protein_seq, compound_x, seq_W, seq_b, c1_Wl, c1_Wr, c1_b, c2_Wl,
           c2_Wr, c2_b, c3_Wl, c3_Wr, c3_b, p1_Wl, p1_Wr, p1_b, p2_Wl, p2_Wr,
           p2_b, p3_Wl, p3_Wr, p3_b, cfc_W, cfc_b, pfc_W, pfc_b, fc1_W,
           fc1_b, out_W, out_b, compound_edge_index, compound_batch,
           protein_edge_index, protein_batch):
    xpad = jnp.pad(protein_seq, ((0, 0), (2, 2), (0, 0)))
    wcat3 = seq_W.transpose(2, 1, 0)
    pad = EPAD - E
    j = jnp.arange(pad, dtype=jnp.int32)
    pad_src = (j * 97) % NTOT
    pad_dst = NTOT + (j % (NACC - NTOT))
    src = jnp.concatenate(
        [compound_edge_index[0], protein_edge_index[0] + NC_NODES, pad_src]
    ).reshape(NW, NCH, CHUNK)
    dst = jnp.concatenate(
        [compound_edge_index[1], protein_edge_index[1] + NC_NODES, pad_dst]
    ).reshape(NW, NCH, CHUNK)

    rs = lambda v: v.reshape(1, -1)

    t1, r1 = _dense_a(xpad, compound_x, wcat3, rs(seq_b), c1_Wl.T, c1_Wr.T,
                      rs(c1_b), p1_Wl.T, p1_Wr.T, rs(p1_b))
    u1 = _psum(_segsum32(t1, src, dst))

    t2 = _combine1(u1, r1)
    r2 = _rproj(t2, c2_Wr.T, rs(c2_b), p2_Wr.T, rs(p2_b), 64)
    u2 = _psum(_segsum32(t2, src, dst))

    x2 = _combine2(u2, r2, c2_Wl.T, p2_Wl.T)
    t3a, t3b, r3 = _split3(x2, c3_Wr.T, rs(c3_b), p3_Wr.T, rs(p3_b))
    ua = _psum(_segsum32(t3a, src, dst))
    ub = _psum(_segsum32(t3b, src, dst))

    cwl3 = c3_Wl.T
    pwl3 = p3_Wl.T
    xc3 = _combine3(ua[0:NC_NODES], ub[0:NC_NODES], r3[0:NC_NODES],
                    cwl3[0:32], cwl3[32:64], NC_NODES)
    xp3 = _combine3(ua[NC_NODES:NTOT], ub[NC_NODES:NTOT], r3[NC_NODES:NTOT],
                    pwl3[0:32], pwl3[32:64], NP_NODES)
    cp = _pool(xc3, compound_batch.reshape(-1, 1), NC_NODES, 8)
    pp = _pool(xp3, protein_batch.reshape(-1, 1), NP_NODES, 8)
    return _head(cp, pp, cfc_W.T, rs(cfc_b), pfc_W.T, rs(pfc_b),
                 fc1_W.T, rs(fc1_b), out_W.T, rs(out_b))
